# baseline (speedup 1.0000x reference)
import functools

import jax
import jax.numpy as jnp
from jax import lax
from jax.experimental import pallas as pl
from jax.experimental.pallas import tpu as pltpu
from jax.experimental.pallas import tpu_sc as plsc

_NUM_EMBEDDINGS = 1000000
_FEATURES = 64
_ROWS, _COLS = 16384, 50

_NC = 2
_NS = 16
_NW = _NC * _NS
_RPW = _ROWS // _NW
_PER_W = _RPW * _COLS
_CHUNK = 128
_NCHUNKS = _PER_W // _CHUNK
_K = 2
_SUPER = _K * _CHUNK
_NSUPER = _NCHUNKS // _K
_NB = 4
_NGROUP = _NSUPER // _NB

_mesh = plsc.VectorSubcoreMesh(core_axis_name="c", subcore_axis_name="s")


@functools.partial(
    pl.kernel,
    mesh=_mesh,
    out_type=jax.ShapeDtypeStruct((_ROWS * _COLS, _FEATURES), jnp.float32),
    scratch_types=[
        pltpu.VMEM((_NCHUNKS, _CHUNK), jnp.int32),
        [pltpu.VMEM((_SUPER, _FEATURES), jnp.float32) for _ in range(_NB)],
        [pltpu.SemaphoreType.DMA for _ in range(_NB)],
        [pltpu.SemaphoreType.DMA for _ in range(_NB)],
    ],
    compiler_params=pltpu.CompilerParams(use_tc_tiling_on_sc=False),
)
def _embed_sc(tok_hbm, table_hbm, out_hbm, idx_v, bufs, gsems, ssems):
    wid = lax.axis_index("s") * _NC + lax.axis_index("c")
    tok_r = tok_hbm
    out_r = out_hbm
    base = wid * _PER_W
    pltpu.sync_copy(tok_r.at[pl.ds(wid * _NCHUNKS, _NCHUNKS)], idx_v)

    def idx_at(j):
        return idx_v.at[j]

    def fire_gathers(g, b):
        for k in range(_K):
            pltpu.async_copy(
                table_hbm.at[idx_at(g * _K + k)],
                bufs[b].at[pl.ds(k * _CHUNK, _CHUNK)],
                gsems[b],
            )

    def drain_gathers(b):
        for k in range(_K):
            pltpu.make_async_copy(
                table_hbm.at[idx_at(k)],
                bufs[b].at[pl.ds(k * _CHUNK, _CHUNK)],
                gsems[b],
            ).wait()

    def drain_scatter(b):
        pltpu.make_async_copy(
            bufs[b], out_r.at[pl.ds(base, _SUPER)], ssems[b]).wait()

    for b in range(_NB):
        fire_gathers(b, b)

    def body(i, _):
        g0 = i * _NB
        for b in range(_NB):
            drain_gathers(b)
            pltpu.async_copy(
                bufs[b],
                out_r.at[pl.ds(base + (g0 + b) * _SUPER, _SUPER)],
                ssems[b],
            )

        @pl.when(i < _NGROUP - 1)
        def _():
            for b in range(_NB):
                drain_scatter(b)
                fire_gathers(g0 + _NB + b, b)

        return 0

    lax.fori_loop(0, _NGROUP, body, 0)
    for b in range(_NB):
        drain_scatter(b)


def kernel(tokens, embedding):
    tok_flat = tokens.astype(jnp.int32).reshape(_NW * _NCHUNKS, _CHUNK)
    out = _embed_sc(tok_flat, embedding)
    return out.reshape(_ROWS, _COLS, _FEATURES)
